# baseline (speedup 1.0000x reference)
import dataclasses

import jax
import jax.numpy as jnp
from jax import lax
from jax.experimental import pallas as pl
from jax.experimental.pallas import tpu as pltpu
from jax.experimental.pallas import tpu_sc as plsc

B = 4096
L = 200
V = 1000000
D = 64
EPS = 1e-5
POOL = 203

NC = 2
NS = 16
NW = NC * NS
BPW = B // NW

LHALF = L // 2
NBUF = 4

NTC = V // 128
STW = 136

_mesh = plsc.VectorSubcoreMesh(core_axis_name="c", subcore_axis_name="s")


def _sc_cp():
    cp = pltpu.CompilerParams()
    if "needs_layout_passes" in pltpu.CompilerParams.__dataclass_fields__:
        cp = dataclasses.replace(cp, needs_layout_passes=False)
    return cp


_UITERS = NTC // NW + 2


def _sc_untile_kernel(tv_hbm, tail_hbm, out_hbm,
                      in0, in1, st0, st1, tail_v,
                      si0, si1, so0, so1):
    ins, sts = (in0, in1), (st0, st1)
    sis, sos = (si0, si1), (so0, so1)
    wid = lax.axis_index("s") * NC + lax.axis_index("c")
    iota = lax.iota(jnp.int32, 16)
    rowv = iota // 2
    colbase = (iota % 2) * 64

    def start_of(k):
        tc = wid + NW * k
        tc = jnp.where(tc < NTC, tc, wid)
        return pl.multiple_of(tc * 128, 128)

    def fire_in(k, s):
        pltpu.make_async_copy(
            tv_hbm.at[:, pl.ds(start_of(k), 128)], ins[s], sis[s]).start()

    def wait_in(s):
        pltpu.make_async_copy(
            tv_hbm.at[:, pl.ds(0, 128)], ins[s], sis[s]).wait()

    def fire_out(k, s):
        pltpu.make_async_copy(
            sts[s].at[pl.ds(0, 64), pl.ds(0, 128)],
            out_hbm.at[pl.ds(pl.multiple_of(start_of(k) // 2, 64), 64)],
            sos[s]).start()

    def wait_out(s):
        pltpu.make_async_copy(
            sts[s].at[pl.ds(0, 64), pl.ds(0, 128)],
            out_hbm.at[pl.ds(0, 64)], sos[s]).wait()

    def compute(s):
        in_v, stage_v = ins[s], sts[s]
        for c in range(8):
            rvec = 8 * c + rowv

            @pl.loop(0, D, step=4)
            def _(f):
                for k in range(4):
                    val = in_v[f + k, pl.ds(16 * c, 16)]
                    plsc.store_scatter(stage_v, [rvec, colbase + (f + k)], val)

    fire_in(0, 0)
    fire_in(1, 1)
    for s in range(2):
        wait_in(s)
        compute(s)
        fire_out(s, s)
        fire_in(s + 2, s)

    @pl.loop(2, _UITERS - 2, step=2)
    def _(k):
        for s in range(2):
            wait_in(s)
            wait_out(s)
            compute(s)
            fire_out(k + s, s)
            fire_in(k + s + 2, s)

    for s in range(2):
        wait_in(s)
        wait_out(s)
        compute(s)
        fire_out(_UITERS - 2 + s, s)
    for s in range(2):
        wait_out(s)

    @pl.when(wid == 0)
    def _():
        pltpu.sync_copy(tail_hbm, tail_v)

        @pl.loop(0, 256)
        def _(k):
            st0[k // 8, pl.ds(16 * (k % 8), 16)] = tail_v[pl.ds(16 * k, 16)]

        pltpu.sync_copy(st0.at[pl.ds(0, 32), pl.ds(0, 128)],
                        out_hbm.at[pl.ds((V - 64) // 2, 32)])


def _sc_untile(table):
    kern = pl.kernel(
        _sc_untile_kernel,
        out_type=jax.ShapeDtypeStruct((V // 2, 128), jnp.float32),
        mesh=_mesh,
        compiler_params=_sc_cp(),
        scratch_types=[
            pltpu.VMEM((D, 128), jnp.float32),
            pltpu.VMEM((D, 128), jnp.float32),
            pltpu.VMEM((64, STW), jnp.float32),
            pltpu.VMEM((64, STW), jnp.float32),
            pltpu.VMEM((4096,), jnp.float32),
            pltpu.SemaphoreType.DMA,
            pltpu.SemaphoreType.DMA,
            pltpu.SemaphoreType.DMA,
            pltpu.SemaphoreType.DMA,
        ],
    )
    return kern(table.T, table[V - 64:].reshape(4096))


def _sc_pool_kernel(plates_hbm, ptab_hbm, out_hbm, pidx_all, out_v, *ring):
    bufs = ring[:NBUF]
    sems = ring[NBUF:]
    wid = lax.axis_index("s") * NC + lax.axis_index("c")
    base = wid * BPW

    pltpu.sync_copy(plates_hbm.at[pl.ds(wid * 2 * BPW, 2 * BPW)], pidx_all)

    def fire(row, b):
        pltpu.make_async_copy(
            ptab_hbm.at[pidx_all.at[2 * row]],
            bufs[b].at[pl.ds(0, LHALF)], sems[b]).start()
        pltpu.make_async_copy(
            ptab_hbm.at[pidx_all.at[2 * row + 1]],
            bufs[b].at[pl.ds(LHALF, LHALF)], sems[b]).start()

    def drain(b):
        pltpu.make_async_copy(
            ptab_hbm.at[pl.ds(0, L)], bufs[b], sems[b]).wait()

    def accum(row, b):
        buf = bufs[b]

        def body(j, acc):
            a0, a1, a2, a3, b0, b1, b2, b3 = acc
            a0 = a0 + buf[j, pl.ds(0, 16)]
            a1 = a1 + buf[j, pl.ds(16, 16)]
            a2 = a2 + buf[j, pl.ds(32, 16)]
            a3 = a3 + buf[j, pl.ds(48, 16)]
            b0 = b0 + buf[j + LHALF, pl.ds(0, 16)]
            b1 = b1 + buf[j + LHALF, pl.ds(16, 16)]
            b2 = b2 + buf[j + LHALF, pl.ds(32, 16)]
            b3 = b3 + buf[j + LHALF, pl.ds(48, 16)]
            return (a0, a1, a2, a3, b0, b1, b2, b3)

        z = jnp.zeros((16,), jnp.float32)
        a0, a1, a2, a3, b0, b1, b2, b3 = lax.fori_loop(
            0, LHALF, body, (z,) * 8, unroll=2)
        out_v[row, pl.ds(0, 16)] = a0 + b0
        out_v[row, pl.ds(16, 16)] = a1 + b1
        out_v[row, pl.ds(32, 16)] = a2 + b2
        out_v[row, pl.ds(48, 16)] = a3 + b3

    for b in range(NBUF):
        fire(b, b)

    @pl.loop(0, BPW - NBUF, step=NBUF)
    def _(i):
        for b in range(NBUF):
            drain(b)
            accum(i + b, b)
            fire(i + b + NBUF, b)

    for b in range(NBUF):
        drain(b)
        accum(BPW - NBUF + b, b)

    pltpu.sync_copy(out_v, out_hbm.at[pl.ds(base, BPW)])


def _sc_pool(plates, ptab_lin):
    kern = pl.kernel(
        _sc_pool_kernel,
        out_type=jax.ShapeDtypeStruct((B, D), jnp.float32),
        mesh=_mesh,
        compiler_params=pltpu.CompilerParams(use_tc_tiling_on_sc=False),
        scratch_types=[
            pltpu.VMEM((2 * BPW, LHALF), jnp.int32),
            pltpu.VMEM((BPW, D), jnp.float32),
        ] + [pltpu.VMEM((L, D), jnp.float32) for _ in range(NBUF)]
          + [pltpu.SemaphoreType.DMA for _ in range(NBUF)],
    )
    return kern(plates.reshape(2 * B, LHALF), ptab_lin)


def _tc_mlp_kernel(x_ref, adv_ref, sig_ref, yr_ref, atab_ref, stab_ref,
                   ytab_ref, lng_ref, lnb_ref, w1_ref, b1_ref, w2_ref,
                   b2_ref, w3_ref, b3_ref, out_ref):
    x = x_ref[...]
    adv = adv_ref[...]
    sig = sig_ref[...]
    yr = yr_ref[...]
    for k in range(2):
        x = x + jnp.where(adv == k, 1.0, 0.0) * atab_ref[k][None, :]
    for k in range(11):
        x = x + jnp.where(sig == k, 1.0, 0.0) * stab_ref[k][None, :]
    for k in range(5):
        x = x + jnp.where(yr == k, 1.0, 0.0) * ytab_ref[k][None, :]
    x = x * (1.0 / POOL)
    mu = jnp.mean(x, axis=1, keepdims=True)
    xc = x - mu
    var = jnp.mean(xc * xc, axis=1, keepdims=True)
    x = xc * lax.rsqrt(var + EPS) * lng_ref[...] + lnb_ref[...]
    h = jnp.dot(x, w1_ref[...], preferred_element_type=jnp.float32)
    h = jnp.maximum(h + b1_ref[...], 0.0)
    h = jnp.dot(h, w2_ref[...], preferred_element_type=jnp.float32)
    h = jnp.maximum(h + b2_ref[...], 0.0)
    out_ref[...] = (
        jnp.dot(h, w3_ref[...], preferred_element_type=jnp.float32)
        + b3_ref[...])


def _tc_mlp(pooled, adv, sig, yr, atab, stab, ytab,
            ln_g, ln_b, W1, b1, W2, b2, W3, b3):
    return pl.pallas_call(
        _tc_mlp_kernel,
        out_shape=jax.ShapeDtypeStruct((B, 1), jnp.float32),
    )(pooled, adv.reshape(B, 1), sig.reshape(B, 1), yr.reshape(B, 1),
      atab, stab, ytab, ln_g.reshape(1, D), ln_b.reshape(1, D),
      W1, b1.reshape(1, 128), W2, b2.reshape(1, 64), W3, b3.reshape(1, 1))


@jax.jit
def kernel(plates, advantages_on_road, significances, years, plate_table,
           adv_table, sig_table, year_table, ln_g, ln_b, W1, b1, W2, b2,
           W3, b3):
    tbl2 = _sc_untile(plate_table)
    pooled = _sc_pool(plates, tbl2.reshape(V, D))
    return _tc_mlp(pooled, advantages_on_road, significances, years,
                   adv_table, sig_table, year_table,
                   ln_g, ln_b, W1, b1, W2, b2, W3, b3)


# speedup vs baseline: 1.0021x; 1.0021x over previous
import dataclasses

import jax
import jax.numpy as jnp
from jax import lax
from jax.experimental import pallas as pl
from jax.experimental.pallas import tpu as pltpu
from jax.experimental.pallas import tpu_sc as plsc

B = 4096
L = 200
V = 1000000
D = 64
EPS = 1e-5
POOL = 203

NC = 2
NS = 16
NW = NC * NS
BPW = B // NW

LHALF = L // 2
NBUF = 4

NTC = V // 128
STW = 138

_mesh = plsc.VectorSubcoreMesh(core_axis_name="c", subcore_axis_name="s")


def _sc_cp():
    cp = pltpu.CompilerParams()
    if "needs_layout_passes" in pltpu.CompilerParams.__dataclass_fields__:
        cp = dataclasses.replace(cp, needs_layout_passes=False)
    return cp


_UITERS = NTC // NW + 2


def _sc_untile_kernel(tv_hbm, tail_hbm, out_hbm,
                      in0, in1, st0, st1, tail_v,
                      si0, si1, so0, so1):
    ins, sts = (in0, in1), (st0, st1)
    sis, sos = (si0, si1), (so0, so1)
    wid = lax.axis_index("s") * NC + lax.axis_index("c")
    iota = lax.iota(jnp.int32, 16)
    rowv = iota // 2
    colbase = (iota % 2) * 64

    def start_of(k):
        tc = wid + NW * k
        tc = jnp.where(tc < NTC, tc, wid)
        return pl.multiple_of(tc * 128, 128)

    def fire_in(k, s):
        pltpu.make_async_copy(
            tv_hbm.at[:, pl.ds(start_of(k), 128)], ins[s], sis[s]).start()

    def wait_in(s):
        pltpu.make_async_copy(
            tv_hbm.at[:, pl.ds(0, 128)], ins[s], sis[s]).wait()

    def fire_out(k, s):
        pltpu.make_async_copy(
            sts[s].at[pl.ds(0, 64), pl.ds(0, 128)],
            out_hbm.at[pl.ds(pl.multiple_of(start_of(k) // 2, 64), 64)],
            sos[s]).start()

    def wait_out(s):
        pltpu.make_async_copy(
            sts[s].at[pl.ds(0, 64), pl.ds(0, 128)],
            out_hbm.at[pl.ds(0, 64)], sos[s]).wait()

    def compute(s):
        in_v, stage_v = ins[s], sts[s]
        for c in range(8):
            rvec = 8 * c + rowv

            @pl.loop(0, D, step=4)
            def _(f):
                for k in range(4):
                    val = in_v[f + k, pl.ds(16 * c, 16)]
                    plsc.store_scatter(stage_v, [rvec, colbase + (f + k)], val)

    fire_in(0, 0)
    fire_in(1, 1)
    for s in range(2):
        wait_in(s)
        compute(s)
        fire_out(s, s)
        fire_in(s + 2, s)

    @pl.loop(2, _UITERS - 2, step=2)
    def _(k):
        for s in range(2):
            wait_in(s)
            wait_out(s)
            compute(s)
            fire_out(k + s, s)
            fire_in(k + s + 2, s)

    for s in range(2):
        wait_in(s)
        wait_out(s)
        compute(s)
        fire_out(_UITERS - 2 + s, s)
    for s in range(2):
        wait_out(s)

    @pl.when(wid == 0)
    def _():
        pltpu.sync_copy(tail_hbm, tail_v)

        @pl.loop(0, 256)
        def _(k):
            st0[k // 8, pl.ds(16 * (k % 8), 16)] = tail_v[pl.ds(16 * k, 16)]

        pltpu.sync_copy(st0.at[pl.ds(0, 32), pl.ds(0, 128)],
                        out_hbm.at[pl.ds((V - 64) // 2, 32)])


def _sc_untile(table):
    kern = pl.kernel(
        _sc_untile_kernel,
        out_type=jax.ShapeDtypeStruct((V // 2, 128), jnp.float32),
        mesh=_mesh,
        compiler_params=_sc_cp(),
        scratch_types=[
            pltpu.VMEM((D, 128), jnp.float32),
            pltpu.VMEM((D, 128), jnp.float32),
            pltpu.VMEM((64, STW), jnp.float32),
            pltpu.VMEM((64, STW), jnp.float32),
            pltpu.VMEM((4096,), jnp.float32),
            pltpu.SemaphoreType.DMA,
            pltpu.SemaphoreType.DMA,
            pltpu.SemaphoreType.DMA,
            pltpu.SemaphoreType.DMA,
        ],
    )
    return kern(table.T, table[V - 64:].reshape(4096))


def _sc_pool_kernel(plates_hbm, ptab_hbm, out_hbm, pidx_all, out_v, *ring):
    bufs = ring[:NBUF]
    sems = ring[NBUF:]
    wid = lax.axis_index("s") * NC + lax.axis_index("c")
    base = wid * BPW

    pltpu.sync_copy(plates_hbm.at[pl.ds(wid * 2 * BPW, 2 * BPW)], pidx_all)

    def fire(row, b):
        pltpu.make_async_copy(
            ptab_hbm.at[pidx_all.at[2 * row]],
            bufs[b].at[pl.ds(0, LHALF)], sems[b]).start()
        pltpu.make_async_copy(
            ptab_hbm.at[pidx_all.at[2 * row + 1]],
            bufs[b].at[pl.ds(LHALF, LHALF)], sems[b]).start()

    def drain(b):
        pltpu.make_async_copy(
            ptab_hbm.at[pl.ds(0, L)], bufs[b], sems[b]).wait()

    def accum(row, b):
        buf = bufs[b]

        def body(j, acc):
            a0, a1, a2, a3, b0, b1, b2, b3 = acc
            a0 = a0 + buf[j, pl.ds(0, 16)]
            a1 = a1 + buf[j, pl.ds(16, 16)]
            a2 = a2 + buf[j, pl.ds(32, 16)]
            a3 = a3 + buf[j, pl.ds(48, 16)]
            b0 = b0 + buf[j + LHALF, pl.ds(0, 16)]
            b1 = b1 + buf[j + LHALF, pl.ds(16, 16)]
            b2 = b2 + buf[j + LHALF, pl.ds(32, 16)]
            b3 = b3 + buf[j + LHALF, pl.ds(48, 16)]
            return (a0, a1, a2, a3, b0, b1, b2, b3)

        z = jnp.zeros((16,), jnp.float32)
        a0, a1, a2, a3, b0, b1, b2, b3 = lax.fori_loop(
            0, LHALF, body, (z,) * 8, unroll=2)
        out_v[row, pl.ds(0, 16)] = a0 + b0
        out_v[row, pl.ds(16, 16)] = a1 + b1
        out_v[row, pl.ds(32, 16)] = a2 + b2
        out_v[row, pl.ds(48, 16)] = a3 + b3

    for b in range(NBUF):
        fire(b, b)

    @pl.loop(0, BPW - NBUF, step=NBUF)
    def _(i):
        for b in range(NBUF):
            drain(b)
            accum(i + b, b)
            fire(i + b + NBUF, b)

    for b in range(NBUF):
        drain(b)
        accum(BPW - NBUF + b, b)

    pltpu.sync_copy(out_v, out_hbm.at[pl.ds(base, BPW)])


def _sc_pool(plates, ptab_lin):
    kern = pl.kernel(
        _sc_pool_kernel,
        out_type=jax.ShapeDtypeStruct((B, D), jnp.float32),
        mesh=_mesh,
        compiler_params=pltpu.CompilerParams(use_tc_tiling_on_sc=False),
        scratch_types=[
            pltpu.VMEM((2 * BPW, LHALF), jnp.int32),
            pltpu.VMEM((BPW, D), jnp.float32),
        ] + [pltpu.VMEM((L, D), jnp.float32) for _ in range(NBUF)]
          + [pltpu.SemaphoreType.DMA for _ in range(NBUF)],
    )
    return kern(plates.reshape(2 * B, LHALF), ptab_lin)


def _tc_mlp_kernel(x_ref, adv_ref, sig_ref, yr_ref, atab_ref, stab_ref,
                   ytab_ref, lng_ref, lnb_ref, w1_ref, b1_ref, w2_ref,
                   b2_ref, w3_ref, b3_ref, out_ref):
    x = x_ref[...]
    adv = adv_ref[...]
    sig = sig_ref[...]
    yr = yr_ref[...]
    for k in range(2):
        x = x + jnp.where(adv == k, 1.0, 0.0) * atab_ref[k][None, :]
    for k in range(11):
        x = x + jnp.where(sig == k, 1.0, 0.0) * stab_ref[k][None, :]
    for k in range(5):
        x = x + jnp.where(yr == k, 1.0, 0.0) * ytab_ref[k][None, :]
    x = x * (1.0 / POOL)
    mu = jnp.mean(x, axis=1, keepdims=True)
    xc = x - mu
    var = jnp.mean(xc * xc, axis=1, keepdims=True)
    x = xc * lax.rsqrt(var + EPS) * lng_ref[...] + lnb_ref[...]
    h = jnp.dot(x, w1_ref[...], preferred_element_type=jnp.float32)
    h = jnp.maximum(h + b1_ref[...], 0.0)
    h = jnp.dot(h, w2_ref[...], preferred_element_type=jnp.float32)
    h = jnp.maximum(h + b2_ref[...], 0.0)
    out_ref[...] = (
        jnp.dot(h, w3_ref[...], preferred_element_type=jnp.float32)
        + b3_ref[...])


def _tc_mlp(pooled, adv, sig, yr, atab, stab, ytab,
            ln_g, ln_b, W1, b1, W2, b2, W3, b3):
    return pl.pallas_call(
        _tc_mlp_kernel,
        out_shape=jax.ShapeDtypeStruct((B, 1), jnp.float32),
    )(pooled, adv.reshape(B, 1), sig.reshape(B, 1), yr.reshape(B, 1),
      atab, stab, ytab, ln_g.reshape(1, D), ln_b.reshape(1, D),
      W1, b1.reshape(1, 128), W2, b2.reshape(1, 64), W3, b3.reshape(1, 1))


@jax.jit
def kernel(plates, advantages_on_road, significances, years, plate_table,
           adv_table, sig_table, year_table, ln_g, ln_b, W1, b1, W2, b2,
           W3, b3):
    tbl2 = _sc_untile(plate_table)
    pooled = _sc_pool(plates, tbl2.reshape(V, D))
    return _tc_mlp(pooled, advantages_on_road, significances, years,
                   adv_table, sig_table, year_table,
                   ln_g, ln_b, W1, b1, W2, b2, W3, b3)


# speedup vs baseline: 1.8866x; 1.8827x over previous
import dataclasses

import jax
import jax.numpy as jnp
from jax import lax
from jax.experimental import pallas as pl
from jax.experimental.pallas import tpu as pltpu
from jax.experimental.pallas import tpu_sc as plsc

B = 4096
L = 200
V = 1000000
D = 64
EPS = 1e-5
POOL = 203

NC = 2
NS = 16
NW = NC * NS
BPW = B // NW

LHALF = L // 2
NBUF = 4

NTC = V // 128
STW = 138

_mesh = plsc.VectorSubcoreMesh(core_axis_name="c", subcore_axis_name="s")


def _sc_cp():
    cp = pltpu.CompilerParams()
    if "needs_layout_passes" in pltpu.CompilerParams.__dataclass_fields__:
        cp = dataclasses.replace(cp, needs_layout_passes=False)
    return cp


def _sc_pool_kernel(plates_hbm, ptab_hbm, out_hbm, pidx_all, out_v, *ring):
    bufs = ring[:NBUF]
    sems = ring[NBUF:]
    wid = lax.axis_index("s") * NC + lax.axis_index("c")
    base = wid * BPW

    pltpu.sync_copy(plates_hbm.at[pl.ds(wid * 2 * BPW, 2 * BPW)], pidx_all)

    def fire(row, b):
        pltpu.make_async_copy(
            ptab_hbm.at[pidx_all.at[2 * row]],
            bufs[b].at[pl.ds(0, LHALF)], sems[b]).start()
        pltpu.make_async_copy(
            ptab_hbm.at[pidx_all.at[2 * row + 1]],
            bufs[b].at[pl.ds(LHALF, LHALF)], sems[b]).start()

    def drain(b):
        pltpu.make_async_copy(
            ptab_hbm.at[pl.ds(0, L)], bufs[b], sems[b]).wait()

    def accum(row, b):
        buf = bufs[b]

        def body(j, acc):
            a0, a1, a2, a3, b0, b1, b2, b3 = acc
            a0 = a0 + buf[j, pl.ds(0, 16)]
            a1 = a1 + buf[j, pl.ds(16, 16)]
            a2 = a2 + buf[j, pl.ds(32, 16)]
            a3 = a3 + buf[j, pl.ds(48, 16)]
            b0 = b0 + buf[j + LHALF, pl.ds(0, 16)]
            b1 = b1 + buf[j + LHALF, pl.ds(16, 16)]
            b2 = b2 + buf[j + LHALF, pl.ds(32, 16)]
            b3 = b3 + buf[j + LHALF, pl.ds(48, 16)]
            return (a0, a1, a2, a3, b0, b1, b2, b3)

        z = jnp.zeros((16,), jnp.float32)
        a0, a1, a2, a3, b0, b1, b2, b3 = lax.fori_loop(
            0, LHALF, body, (z,) * 8, unroll=2)
        out_v[row, pl.ds(0, 16)] = a0 + b0
        out_v[row, pl.ds(16, 16)] = a1 + b1
        out_v[row, pl.ds(32, 16)] = a2 + b2
        out_v[row, pl.ds(48, 16)] = a3 + b3

    for b in range(NBUF):
        fire(b, b)

    @pl.loop(0, BPW - NBUF, step=NBUF)
    def _(i):
        for b in range(NBUF):
            drain(b)
            accum(i + b, b)
            fire(i + b + NBUF, b)

    for b in range(NBUF):
        drain(b)
        accum(BPW - NBUF + b, b)

    pltpu.sync_copy(out_v, out_hbm.at[pl.ds(base, BPW)])


def _sc_pool(plates, ptab_lin):
    kern = pl.kernel(
        _sc_pool_kernel,
        out_type=jax.ShapeDtypeStruct((B, D), jnp.float32),
        mesh=_mesh,
        compiler_params=pltpu.CompilerParams(use_tc_tiling_on_sc=False),
        scratch_types=[
            pltpu.VMEM((2 * BPW, LHALF), jnp.int32),
            pltpu.VMEM((BPW, D), jnp.float32),
        ] + [pltpu.VMEM((L, D), jnp.float32) for _ in range(NBUF)]
          + [pltpu.SemaphoreType.DMA for _ in range(NBUF)],
    )
    return kern(plates.reshape(2 * B, LHALF), ptab_lin)


def _tc_mlp_kernel(x_ref, adv_ref, sig_ref, yr_ref, atab_ref, stab_ref,
                   ytab_ref, lng_ref, lnb_ref, w1_ref, b1_ref, w2_ref,
                   b2_ref, w3_ref, b3_ref, out_ref):
    x = x_ref[...]
    adv = adv_ref[...]
    sig = sig_ref[...]
    yr = yr_ref[...]
    for k in range(2):
        x = x + jnp.where(adv == k, 1.0, 0.0) * atab_ref[k][None, :]
    for k in range(11):
        x = x + jnp.where(sig == k, 1.0, 0.0) * stab_ref[k][None, :]
    for k in range(5):
        x = x + jnp.where(yr == k, 1.0, 0.0) * ytab_ref[k][None, :]
    x = x * (1.0 / POOL)
    mu = jnp.mean(x, axis=1, keepdims=True)
    xc = x - mu
    var = jnp.mean(xc * xc, axis=1, keepdims=True)
    x = xc * lax.rsqrt(var + EPS) * lng_ref[...] + lnb_ref[...]
    h = jnp.dot(x, w1_ref[...], preferred_element_type=jnp.float32)
    h = jnp.maximum(h + b1_ref[...], 0.0)
    h = jnp.dot(h, w2_ref[...], preferred_element_type=jnp.float32)
    h = jnp.maximum(h + b2_ref[...], 0.0)
    out_ref[...] = (
        jnp.dot(h, w3_ref[...], preferred_element_type=jnp.float32)
        + b3_ref[...])


def _tc_mlp(pooled, adv, sig, yr, atab, stab, ytab,
            ln_g, ln_b, W1, b1, W2, b2, W3, b3):
    return pl.pallas_call(
        _tc_mlp_kernel,
        out_shape=jax.ShapeDtypeStruct((B, 1), jnp.float32),
    )(pooled, adv.reshape(B, 1), sig.reshape(B, 1), yr.reshape(B, 1),
      atab, stab, ytab, ln_g.reshape(1, D), ln_b.reshape(1, D),
      W1, b1.reshape(1, 128), W2, b2.reshape(1, 64), W3, b3.reshape(1, 1))


@jax.jit
def kernel(plates, advantages_on_road, significances, years, plate_table,
           adv_table, sig_table, year_table, ln_g, ln_b, W1, b1, W2, b2,
           W3, b3):
    pooled = _sc_pool(plates, plate_table)
    return _tc_mlp(pooled, advantages_on_road, significances, years,
                   adv_table, sig_table, year_table,
                   ln_g, ln_b, W1, b1, W2, b2, W3, b3)
